# Initial kernel scaffold; baseline (speedup 1.0000x reference)
import functools

import jax
import jax.numpy as jnp
from jax import lax
from jax.experimental import pallas as pl
from jax.experimental.pallas import tpu as pltpu
from jax.experimental.pallas import tpu_sc as plsc

IMAGE_C = 3
IMAGE_H = 32
IMAGE_W = 32
D_MODEL = 1024

_L = 16
_NCHUNK = D_MODEL // _L
_BLOCK = IMAGE_W * IMAGE_C
_HALF = _BLOCK // 2
_CPH = IMAGE_W // 2
_T_OUT = 1 + IMAGE_H * IMAGE_W * IMAGE_C


def _make_sc_kernel():
    mesh = plsc.VectorSubcoreMesh(core_axis_name="c", subcore_axis_name="s")
    nc = 2

    @functools.partial(
        pl.kernel,
        mesh=mesh,
        out_type=jax.ShapeDtypeStruct((_T_OUT, D_MODEL), jnp.float32),
        scratch_types=[
            pltpu.VMEM((IMAGE_W, D_MODEL), jnp.float32),
            pltpu.VMEM((IMAGE_C, D_MODEL), jnp.float32),
            pltpu.VMEM((1, D_MODEL), jnp.float32),
            pltpu.VMEM((_HALF, D_MODEL), jnp.float32),
        ],
    )
    def sc_kernel(row_hbm, col_hbm, chn_hbm, out_hbm, col_v, rk_v, row_v, out_v):
        wid = lax.axis_index("s") * nc + lax.axis_index("c")
        r = wid

        pltpu.sync_copy(row_hbm.at[pl.ds(r, 1)], row_v)
        pltpu.sync_copy(col_hbm, col_v)
        pltpu.sync_copy(chn_hbm, rk_v)

        def rk_body(i, carry):
            sl = pl.ds(i * _L, _L)
            rv = row_v[0, sl]
            for k in range(IMAGE_C):
                rk_v[k, sl] = rk_v[k, sl] + rv
            return carry

        lax.fori_loop(0, _NCHUNK, rk_body, 0)

        @pl.when(wid == 0)
        def _():
            zeros = jnp.zeros((_L,), jnp.float32)

            def z_body(i, carry):
                row_v[0, pl.ds(i * _L, _L)] = zeros
                return carry

            lax.fori_loop(0, _NCHUNK, z_body, 0)
            pltpu.sync_copy(row_v, out_hbm.at[pl.ds(0, 1)])

        for half in range(2):
            def c_body(i, carry, half=half):
                sl = pl.ds(i * _L, _L)
                rk0 = rk_v[0, sl]
                rk1 = rk_v[1, sl]
                rk2 = rk_v[2, sl]
                for cl in range(_CPH):
                    cv = col_v[half * _CPH + cl, sl]
                    out_v[3 * cl + 0, sl] = cv + rk0
                    out_v[3 * cl + 1, sl] = cv + rk1
                    out_v[3 * cl + 2, sl] = cv + rk2
                return carry

            lax.fori_loop(0, _NCHUNK, c_body, 0)
            pltpu.sync_copy(
                out_v, out_hbm.at[pl.ds(1 + r * _BLOCK + half * _HALF, _HALF)]
            )

    return sc_kernel


def kernel(T, row_w, col_w, chn_w):
    return _make_sc_kernel()(row_w, col_w, chn_w)


# speedup vs baseline: 1.0689x; 1.0689x over previous
import functools

import jax
import jax.numpy as jnp
from jax import lax
from jax.experimental import pallas as pl
from jax.experimental.pallas import tpu as pltpu
from jax.experimental.pallas import tpu_sc as plsc

IMAGE_C = 3
IMAGE_H = 32
IMAGE_W = 32
D_MODEL = 1024

_L = 16
_NCHUNK = D_MODEL // _L
_BLOCK = IMAGE_W * IMAGE_C
_HALF = _BLOCK // 2
_CPH = IMAGE_W // 2
_T_OUT = 1 + IMAGE_H * IMAGE_W * IMAGE_C


def _make_sc_kernel():
    mesh = plsc.VectorSubcoreMesh(core_axis_name="c", subcore_axis_name="s")
    nc = 2

    @functools.partial(
        pl.kernel,
        mesh=mesh,
        out_type=jax.ShapeDtypeStruct((_T_OUT, D_MODEL), jnp.float32),
        compiler_params=pltpu.CompilerParams(use_tc_tiling_on_sc=False),
        scratch_types=[
            pltpu.VMEM((IMAGE_W, D_MODEL), jnp.float32),
            pltpu.VMEM((IMAGE_C, D_MODEL), jnp.float32),
            pltpu.VMEM((1, D_MODEL), jnp.float32),
            pltpu.VMEM((_HALF, D_MODEL), jnp.float32),
        ],
    )
    def sc_kernel(row_hbm, col_hbm, chn_hbm, out_hbm, col_v, rk_v, row_v, out_v):
        wid = lax.axis_index("s") * nc + lax.axis_index("c")
        r = wid

        pltpu.sync_copy(row_hbm.at[pl.ds(r, 1)], row_v)
        pltpu.sync_copy(col_hbm, col_v)
        pltpu.sync_copy(chn_hbm, rk_v)

        def rk_body(i, carry):
            sl = pl.ds(i * _L, _L)
            rv = row_v[0, sl]
            for k in range(IMAGE_C):
                rk_v[k, sl] = rk_v[k, sl] + rv
            return carry

        lax.fori_loop(0, _NCHUNK, rk_body, 0)

        @pl.when(wid == 0)
        def _():
            zeros = jnp.zeros((_L,), jnp.float32)

            def z_body(i, carry):
                row_v[0, pl.ds(i * _L, _L)] = zeros
                return carry

            lax.fori_loop(0, _NCHUNK, z_body, 0)
            pltpu.sync_copy(row_v, out_hbm.at[pl.ds(0, 1)])

        for half in range(2):
            def c_body(i, carry, half=half):
                sl = pl.ds(i * _L, _L)
                rk0 = rk_v[0, sl]
                rk1 = rk_v[1, sl]
                rk2 = rk_v[2, sl]
                for cl in range(_CPH):
                    cv = col_v[half * _CPH + cl, sl]
                    out_v[3 * cl + 0, sl] = cv + rk0
                    out_v[3 * cl + 1, sl] = cv + rk1
                    out_v[3 * cl + 2, sl] = cv + rk2
                return carry

            lax.fori_loop(0, _NCHUNK, c_body, 0)
            pltpu.sync_copy(
                out_v, out_hbm.at[pl.ds(1 + r * _BLOCK + half * _HALF, _HALF)]
            )

    return sc_kernel


def kernel(T, row_w, col_w, chn_w):
    return _make_sc_kernel()(row_w, col_w, chn_w)


# speedup vs baseline: 1.2843x; 1.2016x over previous
import functools

import jax
import jax.numpy as jnp
from jax import lax
from jax.experimental import pallas as pl
from jax.experimental.pallas import tpu as pltpu
from jax.experimental.pallas import tpu_sc as plsc

IMAGE_C = 3
IMAGE_H = 32
IMAGE_W = 32
D_MODEL = 1024

_L = 16
_NCHUNK = D_MODEL // _L
_BLOCK = IMAGE_W * IMAGE_C
_HALF = _BLOCK // 2
_T_OUT = 1 + IMAGE_H * IMAGE_W * IMAGE_C


def _half_groups(half):
    groups = {}
    for j in range(_HALF * half, _HALF * half + _HALF):
        if j == 0:
            continue
        c, k = divmod(j - 1, 3)
        groups.setdefault(c, []).append((j - _HALF * half, k))
    return groups


def _make_sc_kernel():
    mesh = plsc.VectorSubcoreMesh(core_axis_name="c", subcore_axis_name="s")
    nc = 2

    @functools.partial(
        pl.kernel,
        mesh=mesh,
        out_type=jax.ShapeDtypeStruct((_T_OUT, D_MODEL), jnp.float32),
        scratch_types=[
            pltpu.VMEM((IMAGE_H, D_MODEL), jnp.float32),
            pltpu.VMEM((IMAGE_W, D_MODEL), jnp.float32),
            pltpu.VMEM((IMAGE_C, D_MODEL), jnp.float32),
            pltpu.VMEM((IMAGE_C, D_MODEL), jnp.float32),
            pltpu.VMEM((1, D_MODEL), jnp.float32),
            pltpu.VMEM((_HALF, D_MODEL), jnp.float32),
        ],
    )
    def sc_kernel(row_hbm, col_hbm, chn_hbm, out_hbm,
                  rowt_v, col_v, chn_v, rk_v, last_v, out_v):
        wid = lax.axis_index("s") * nc + lax.axis_index("c")
        w = wid
        wprev = lax.max(w - 1, 0)
        is_first = w == 0

        pltpu.sync_copy(row_hbm, rowt_v)
        pltpu.sync_copy(col_hbm, col_v)
        pltpu.sync_copy(chn_hbm, chn_v)

        def rk_body(i, carry):
            sl = pl.ds(i * _L, _L)
            rv = rowt_v[w, sl]
            for k in range(IMAGE_C):
                rk_v[k, sl] = chn_v[k, sl] + rv
            return carry

        lax.fori_loop(0, _NCHUNK, rk_body, 0)

        zeros = jnp.zeros((_L,), jnp.float32)
        for half in range(2):
            groups = _half_groups(half)

            def c_body(i, carry, half=half, groups=groups):
                sl = pl.ds(i * _L, _L)
                rk = [rk_v[0, sl], rk_v[1, sl], rk_v[2, sl]]
                if half == 0:
                    tail = rowt_v[wprev, sl] + col_v[IMAGE_W - 1, sl]
                    tail = tail + chn_v[IMAGE_C - 1, sl]
                    out_v[0, sl] = jnp.where(is_first, zeros, tail)
                for c, rows in groups.items():
                    cv = col_v[c, sl]
                    for lj, k in rows:
                        out_v[lj, sl] = cv + rk[k]
                return carry

            lax.fori_loop(0, _NCHUNK, c_body, 0)
            pltpu.sync_copy(
                out_v, out_hbm.at[pl.ds(w * _BLOCK + half * _HALF, _HALF)]
            )

        @pl.when(wid == IMAGE_H - 1)
        def _():
            def l_body(i, carry):
                sl = pl.ds(i * _L, _L)
                last_v[0, sl] = col_v[IMAGE_W - 1, sl] + rk_v[IMAGE_C - 1, sl]
                return carry

            lax.fori_loop(0, _NCHUNK, l_body, 0)
            pltpu.sync_copy(last_v, out_hbm.at[pl.ds(_T_OUT - 1, 1)])

    return sc_kernel


def kernel(T, row_w, col_w, chn_w):
    return _make_sc_kernel()(row_w, col_w, chn_w)


# speedup vs baseline: 1.5160x; 1.1803x over previous
import functools

import jax
import jax.numpy as jnp
from jax import lax
from jax.experimental import pallas as pl
from jax.experimental.pallas import tpu as pltpu
from jax.experimental.pallas import tpu_sc as plsc

IMAGE_C = 3
IMAGE_H = 32
IMAGE_W = 32
D_MODEL = 1024

_L = 16
_NCHUNK = D_MODEL // _L
_BLOCK = IMAGE_W * IMAGE_C
_NQ = 4
_QROWS = _BLOCK // _NQ
_RWIN = 16
_T_OUT = 1 + IMAGE_H * IMAGE_W * IMAGE_C


def _quarter_groups(q):
    groups = {}
    for j in range(_QROWS * q, _QROWS * (q + 1)):
        if j == 0:
            continue
        c, k = divmod(j - 1, 3)
        groups.setdefault(c, []).append((j - _QROWS * q, k))
    return groups


def _make_sc_kernel():
    mesh = plsc.VectorSubcoreMesh(core_axis_name="c", subcore_axis_name="s")
    nc = 2

    @functools.partial(
        pl.kernel,
        mesh=mesh,
        out_type=jax.ShapeDtypeStruct((_T_OUT, D_MODEL), jnp.float32),
        scratch_types=[
            pltpu.VMEM((_RWIN, D_MODEL), jnp.float32),
            pltpu.VMEM((IMAGE_W, D_MODEL), jnp.float32),
            pltpu.VMEM((IMAGE_C, D_MODEL), jnp.float32),
            pltpu.VMEM((IMAGE_C, D_MODEL), jnp.float32),
            pltpu.VMEM((1, D_MODEL), jnp.float32),
            pltpu.VMEM((2, _QROWS, D_MODEL), jnp.float32),
            pltpu.SemaphoreType.DMA,
            pltpu.SemaphoreType.DMA,
            pltpu.SemaphoreType.DMA,
        ],
    )
    def sc_kernel(row_hbm, col_hbm, chn_hbm, out_hbm,
                  rowt_v, col_v, chn_v, rk_v, last_v, out_v,
                  sem_s, sem_a, sem_b):
        wid = lax.axis_index("s") * nc + lax.axis_index("c")
        w = wid
        wprev = lax.max(w - 1, 0)
        base = (wprev // 8) * 8
        lw = w - base
        lwprev = wprev - base
        is_first = w == 0

        h1 = pltpu.make_async_copy(row_hbm.at[pl.ds(base, _RWIN)], rowt_v, sem_s)
        h2 = pltpu.make_async_copy(col_hbm, col_v, sem_s)
        h3 = pltpu.make_async_copy(chn_hbm, chn_v, sem_s)
        h1.start()
        h2.start()
        h3.start()
        h1.wait()
        h2.wait()
        h3.wait()

        def rk_body(i, carry):
            sl = pl.ds(i * _L, _L)
            rv = rowt_v[lw, sl]
            for k in range(IMAGE_C):
                rk_v[k, sl] = chn_v[k, sl] + rv
            return carry

        lax.fori_loop(0, _NCHUNK, rk_body, 0)

        zeros = jnp.zeros((_L,), jnp.float32)
        sems = [sem_a, sem_b]
        handles = [None, None]
        for q in range(_NQ):
            b = q % 2
            if handles[b] is not None:
                handles[b].wait()
            groups = _quarter_groups(q)

            def c_body(i, carry, b=b, q=q, groups=groups):
                sl = pl.ds(i * _L, _L)
                rk = [rk_v[0, sl], rk_v[1, sl], rk_v[2, sl]]
                if q == 0:
                    tail = rowt_v[lwprev, sl] + col_v[IMAGE_W - 1, sl]
                    tail = tail + chn_v[IMAGE_C - 1, sl]
                    out_v[b, 0, sl] = jnp.where(is_first, zeros, tail)
                for c, rows in groups.items():
                    cv = col_v[c, sl]
                    for lj, k in rows:
                        out_v[b, lj, sl] = cv + rk[k]
                return carry

            lax.fori_loop(0, _NCHUNK, c_body, 0)
            handles[b] = pltpu.make_async_copy(
                out_v.at[b],
                out_hbm.at[pl.ds(w * _BLOCK + q * _QROWS, _QROWS)],
                sems[b],
            )
            handles[b].start()

        @pl.when(wid == IMAGE_H - 1)
        def _():
            def l_body(i, carry):
                sl = pl.ds(i * _L, _L)
                last_v[0, sl] = col_v[IMAGE_W - 1, sl] + rk_v[IMAGE_C - 1, sl]
                return carry

            lax.fori_loop(0, _NCHUNK, l_body, 0)
            pltpu.sync_copy(last_v, out_hbm.at[pl.ds(_T_OUT - 1, 1)])

        handles[0].wait()
        handles[1].wait()

    return sc_kernel


def kernel(T, row_w, col_w, chn_w):
    return _make_sc_kernel()(row_w, col_w, chn_w)


# speedup vs baseline: 1.5871x; 1.0469x over previous
import functools

import jax
import jax.numpy as jnp
from jax import lax
from jax.experimental import pallas as pl
from jax.experimental.pallas import tpu as pltpu
from jax.experimental.pallas import tpu_sc as plsc

IMAGE_C = 3
IMAGE_H = 32
IMAGE_W = 32
D_MODEL = 1024

_L = 16
_NCHUNK = D_MODEL // _L
_BLOCK = IMAGE_W * IMAGE_C
_NQ = 4
_QROWS = _BLOCK // _NQ
_RWIN = 16
_T_OUT = 1 + IMAGE_H * IMAGE_W * IMAGE_C


def _quarter_groups(q):
    groups = {}
    for j in range(_QROWS * q, _QROWS * (q + 1)):
        if j == 0:
            continue
        c, k = divmod(j - 1, 3)
        groups.setdefault(c, []).append((j - _QROWS * q, k))
    return groups


def _make_sc_kernel():
    mesh = plsc.VectorSubcoreMesh(core_axis_name="c", subcore_axis_name="s")
    nc = 2

    @functools.partial(
        pl.kernel,
        mesh=mesh,
        out_type=jax.ShapeDtypeStruct((_T_OUT, D_MODEL), jnp.float32),
        scratch_types=[
            pltpu.VMEM((_RWIN, D_MODEL), jnp.float32),
            pltpu.VMEM((IMAGE_W, D_MODEL), jnp.float32),
            pltpu.VMEM((IMAGE_C, D_MODEL), jnp.float32),
            pltpu.VMEM((IMAGE_C, D_MODEL), jnp.float32),
            pltpu.VMEM((1, D_MODEL), jnp.float32),
            pltpu.VMEM((2, _QROWS, D_MODEL), jnp.float32),
            pltpu.SemaphoreType.DMA,
            pltpu.SemaphoreType.DMA,
            pltpu.SemaphoreType.DMA,
        ],
    )
    def sc_kernel(row_hbm, col_hbm, chn_hbm, out_hbm,
                  rowt_v, col_v, chn_v, rk_v, last_v, out_v,
                  sem_s, sem_a, sem_b):
        wid = lax.axis_index("s") * nc + lax.axis_index("c")
        w = wid
        wprev = lax.max(w - 1, 0)
        base = (wprev // 8) * 8
        lw = w - base
        lwprev = wprev - base
        is_first = w == 0

        h1 = pltpu.make_async_copy(row_hbm.at[pl.ds(base, _RWIN)], rowt_v, sem_s)
        h2 = pltpu.make_async_copy(col_hbm, col_v, sem_s)
        h3 = pltpu.make_async_copy(chn_hbm, chn_v, sem_s)
        h1.start()
        h2.start()
        h3.start()
        h1.wait()
        h2.wait()
        h3.wait()

        @plsc.parallel_loop(0, _NCHUNK, unroll=4)
        def _(i):
            sl = pl.ds(i * _L, _L)
            rv = rowt_v[lw, sl]
            for k in range(IMAGE_C):
                rk_v[k, sl] = chn_v[k, sl] + rv

        zeros = jnp.zeros((_L,), jnp.float32)
        sems = [sem_a, sem_b]
        handles = [None, None]
        for q in range(_NQ):
            b = q % 2
            if handles[b] is not None:
                handles[b].wait()
            groups = _quarter_groups(q)

            @plsc.parallel_loop(0, _NCHUNK, unroll=2)
            def _(i, b=b, q=q, groups=groups):
                sl = pl.ds(i * _L, _L)
                rk = [rk_v[0, sl], rk_v[1, sl], rk_v[2, sl]]
                if q == 0:
                    tail = rowt_v[lwprev, sl] + col_v[IMAGE_W - 1, sl]
                    tail = tail + chn_v[IMAGE_C - 1, sl]
                    out_v[b, 0, sl] = jnp.where(is_first, zeros, tail)
                for c, rows in groups.items():
                    cv = col_v[c, sl]
                    for lj, k in rows:
                        out_v[b, lj, sl] = cv + rk[k]
            handles[b] = pltpu.make_async_copy(
                out_v.at[b],
                out_hbm.at[pl.ds(w * _BLOCK + q * _QROWS, _QROWS)],
                sems[b],
            )
            handles[b].start()

        @pl.when(wid == IMAGE_H - 1)
        def _():
            @plsc.parallel_loop(0, _NCHUNK, unroll=4)
            def _(i):
                sl = pl.ds(i * _L, _L)
                last_v[0, sl] = col_v[IMAGE_W - 1, sl] + rk_v[IMAGE_C - 1, sl]

            pltpu.sync_copy(last_v, out_hbm.at[pl.ds(_T_OUT - 1, 1)])

        handles[0].wait()
        handles[1].wait()

    return sc_kernel


def kernel(T, row_w, col_w, chn_w):
    return _make_sc_kernel()(row_w, col_w, chn_w)
